# baseline (speedup 1.0000x reference)
import functools

import jax
import jax.numpy as jnp
from jax.experimental import pallas as pl

_S = 2
_BI = 200


def _support_body(x_ref, w1_ref, out_ref):
    out_ref[...] = jnp.dot(x_ref[...], w1_ref[...],
                           preferred_element_type=jnp.float32)


def _layer1_body(*refs, s):
    sup_ref, b1_ref, wc_ref, out_ref = refs[s], refs[s + 1], refs[s + 2], refs[s + 3]
    for j in range(s):
        h = jnp.dot(refs[j][...], sup_ref[...], preferred_element_type=jnp.float32)
        h = jnp.maximum(h + b1_ref[...], 0.0)
        out_ref[j] = jnp.dot(h, wc_ref[...], preferred_element_type=jnp.float32)


def _layer2_body(*refs, s, nclass):
    s2_ref, bc_ref, eps_ref, out_ref = refs[s], refs[s + 1], refs[s + 2], refs[s + 3]
    for j in range(s):
        acc = jnp.dot(refs[j][...], s2_ref[...], preferred_element_type=jnp.float32)
        acc = acc + bc_ref[...]
        mean = acc[:, :nclass]
        logstd = acc[:, nclass:]
        z = eps_ref[j] * jnp.exp(logstd) + mean
        m = jnp.max(z, axis=1, keepdims=True)
        zs = z - m
        lse = jnp.log(jnp.sum(jnp.exp(zs), axis=1, keepdims=True))
        out_ref[j] = zs - lse


def kernel(x, adj, W1, b1, W11, b11, W12, b12):
    n, nfeat = x.shape
    nhid = W1.shape[1]
    nclass = W11.shape[1]

    if n % (_S * _BI) == 0:
        s, bi = _S, _BI
    else:
        s, bi = 1, 8
    g = n // s
    nb = g // bi
    nc2 = 2 * nclass

    wc = jnp.concatenate([W11, W12], axis=1)
    bc = jnp.concatenate([b11, b12])[None, :]
    b1r = b1[None, :]
    eps = jax.random.normal(jax.random.key(42), (n, nclass), dtype=jnp.float32)
    eps3 = eps.reshape(s, g, nclass)

    support = pl.pallas_call(
        _support_body,
        grid=(n // (s * bi),),
        in_specs=[
            pl.BlockSpec((s * bi, nfeat), lambda i: (i, 0)),
            pl.BlockSpec((nfeat, nhid), lambda i: (0, 0)),
        ],
        out_specs=pl.BlockSpec((s * bi, nhid), lambda i: (i, 0)),
        out_shape=jax.ShapeDtypeStruct((n, nhid), jnp.float32),
    )(x, W1)

    def adj_spec(j):
        return pl.BlockSpec((bi, n), lambda i, j=j: (i + j * nb, 0))

    s2_3 = pl.pallas_call(
        functools.partial(_layer1_body, s=s),
        grid=(nb,),
        in_specs=(
            [adj_spec(j) for j in range(s)]
            + [
                pl.BlockSpec((n, nhid), lambda i: (0, 0)),
                pl.BlockSpec((1, nhid), lambda i: (0, 0)),
                pl.BlockSpec((nhid, nc2), lambda i: (0, 0)),
            ]
        ),
        out_specs=pl.BlockSpec((s, bi, nc2), lambda i: (0, i, 0)),
        out_shape=jax.ShapeDtypeStruct((s, g, nc2), jnp.float32),
    )(*([adj] * s), support, b1r, wc)
    s2 = s2_3.reshape(n, nc2)

    out3 = pl.pallas_call(
        functools.partial(_layer2_body, s=s, nclass=nclass),
        grid=(nb,),
        in_specs=(
            [adj_spec(j) for j in range(s)]
            + [
                pl.BlockSpec((n, nc2), lambda i: (0, 0)),
                pl.BlockSpec((1, nc2), lambda i: (0, 0)),
                pl.BlockSpec((s, bi, nclass), lambda i: (0, i, 0)),
            ]
        ),
        out_specs=pl.BlockSpec((s, bi, nclass), lambda i: (0, i, 0)),
        out_shape=jax.ShapeDtypeStruct((s, g, nclass), jnp.float32),
    )(*([adj] * s), s2, bc, eps3)

    return out3.reshape(n, nclass)


# speedup vs baseline: 1.0387x; 1.0387x over previous
import functools

import jax
import jax.numpy as jnp
from jax.experimental import pallas as pl
from jax.experimental.pallas import tpu as pltpu

_PAR = pltpu.CompilerParams(dimension_semantics=("parallel",))


def _support_body(x_ref, w1_ref, out_ref):
    out_ref[...] = jnp.dot(x_ref[...], w1_ref[...],
                           preferred_element_type=jnp.float32)


def _layer1_body(adj_ref, sup_ref, b1_ref, wc_ref, out_ref):
    h = jnp.dot(adj_ref[...], sup_ref[...], preferred_element_type=jnp.float32)
    h = jnp.maximum(h + b1_ref[...], 0.0)
    out_ref[...] = jnp.dot(h, wc_ref[...], preferred_element_type=jnp.float32)


def _layer2_body(adj_ref, s2_ref, bc_ref, eps_ref, out_ref, *, nclass):
    acc = jnp.dot(adj_ref[...], s2_ref[...], preferred_element_type=jnp.float32)
    acc = acc + bc_ref[...]
    mean = acc[:, :nclass]
    logstd = acc[:, nclass:]
    z = eps_ref[...] * jnp.exp(logstd) + mean
    m = jnp.max(z, axis=1, keepdims=True)
    zs = z - m
    lse = jnp.log(jnp.sum(jnp.exp(zs), axis=1, keepdims=True))
    out_ref[...] = zs - lse


def kernel(x, adj, W1, b1, W11, b11, W12, b12):
    n, nfeat = x.shape
    nhid = W1.shape[1]
    nclass = W11.shape[1]

    bi = 400 if n % 400 == 0 else 8
    grid = (n // bi,)
    nc2 = 2 * nclass

    wc = jnp.concatenate([W11, W12], axis=1)
    bc = jnp.concatenate([b11, b12])[None, :]
    b1r = b1[None, :]
    eps = jax.random.normal(jax.random.key(42), (n, nclass), dtype=jnp.float32)

    support = pl.pallas_call(
        _support_body,
        grid=grid,
        in_specs=[
            pl.BlockSpec((bi, nfeat), lambda i: (i, 0)),
            pl.BlockSpec((nfeat, nhid), lambda i: (0, 0)),
        ],
        out_specs=pl.BlockSpec((bi, nhid), lambda i: (i, 0)),
        out_shape=jax.ShapeDtypeStruct((n, nhid), jnp.float32),
        compiler_params=_PAR,
    )(x, W1)

    s2 = pl.pallas_call(
        _layer1_body,
        grid=grid,
        in_specs=[
            pl.BlockSpec((bi, n), lambda i: (i, 0)),
            pl.BlockSpec((n, nhid), lambda i: (0, 0)),
            pl.BlockSpec((1, nhid), lambda i: (0, 0)),
            pl.BlockSpec((nhid, nc2), lambda i: (0, 0)),
        ],
        out_specs=pl.BlockSpec((bi, nc2), lambda i: (i, 0)),
        out_shape=jax.ShapeDtypeStruct((n, nc2), jnp.float32),
        compiler_params=_PAR,
    )(adj, support, b1r, wc)

    out = pl.pallas_call(
        functools.partial(_layer2_body, nclass=nclass),
        grid=grid,
        in_specs=[
            pl.BlockSpec((bi, n), lambda i: (i, 0)),
            pl.BlockSpec((n, nc2), lambda i: (0, 0)),
            pl.BlockSpec((1, nc2), lambda i: (0, 0)),
            pl.BlockSpec((bi, nclass), lambda i: (i, 0)),
        ],
        out_specs=pl.BlockSpec((bi, nclass), lambda i: (i, 0)),
        out_shape=jax.ShapeDtypeStruct((n, nclass), jnp.float32),
        compiler_params=_PAR,
    )(adj, s2, bc, eps)

    return out
